# baseline (speedup 1.0000x reference)
import jax
import jax.numpy as jnp
from jax import lax
from jax.experimental import pallas as pl
from jax.experimental.pallas import tpu as pltpu

_B = 16
_G = 128
_K = 32
_R = 64
_L = 128
_N = _R * _L


def _fps_body(x_ref, c_ref, dv_ref):
    iota2 = (lax.broadcasted_iota(jnp.int32, (_R, _L), 0) * _L
             + lax.broadcasted_iota(jnp.int32, (_R, _L), 1))
    subl = lax.broadcasted_iota(jnp.int32, (8, _L), 0)
    lane = lax.broadcasted_iota(jnp.int32, (8, _L), 1)

    dv_ref[:] = jnp.full((_B, _R, _L), 1e10, jnp.float32)

    def step(s, fars):
        new_fars = []
        for b in range(_B):
            x0 = x_ref[b, 0]
            x1 = x_ref[b, 1]
            x2 = x_ref[b, 2]
            far = fars[b]
            oh = iota2 == far
            c0 = jnp.sum(jnp.where(oh, x0, 0.0))
            c1 = jnp.sum(jnp.where(oh, x1, 0.0))
            c2 = jnp.sum(jnp.where(oh, x2, 0.0))
            d0 = x0 - c0
            d1 = x1 - c1
            d2 = x2 - c2
            d = d0 * d0 + d1 * d1 + d2 * d2
            dv = jnp.minimum(dv_ref[b], d)
            dv_ref[b] = dv
            m = jnp.max(dv)
            new_fars.append(jnp.min(jnp.where(dv == m, iota2, _N)))
            crow = jnp.where(subl == 0, c0, jnp.where(subl == 1, c1, c2))
            c_ref[b] = jnp.where(lane == s, crow, c_ref[b])
        return tuple(new_fars)

    lax.fori_loop(0, _G, step, tuple(jnp.array(0, jnp.int32)
                                     for _ in range(_B)))


_P = 4


def _topk_body(xr_ref, c_ref, idx_ref, d_ref):
    for p in range(_P):
        c0a = c_ref[p, :, 0:1]
        c1a = c_ref[p, :, 1:2]
        c2a = c_ref[p, :, 2:3]
        x0r = xr_ref[p, 0:1, :]
        x1r = xr_ref[p, 1:2, :]
        x2r = xr_ref[p, 2:3, :]
        e0 = c0a - x0r
        e1 = c1a - x1r
        e2 = c2a - x2r
        d_ref[p] = jnp.sqrt(e0 * e0 + e1 * e1 + e2 * e2)

    iota_l = lax.broadcasted_iota(jnp.int32, (1, _N), 1)
    iota_k = lax.broadcasted_iota(jnp.int32, (_G, _K), 1)

    m0 = tuple(jnp.min(d_ref[p], axis=1, keepdims=True) for p in range(_P))

    def topk_step(j, carry):
        accs, ms = carry
        new_accs, new_ms = [], []
        for p in range(_P):
            dm = d_ref[p]
            m = ms[p]
            sel = jnp.min(jnp.where(dm == m, iota_l, _N),
                          axis=1, keepdims=True)
            nd = jnp.where(iota_l == sel, jnp.inf, dm)
            d_ref[p] = nd
            new_ms.append(jnp.min(nd, axis=1, keepdims=True))
            new_accs.append(jnp.where(iota_k == j, sel, accs[p]))
        return tuple(new_accs), tuple(new_ms)

    accs, _ = lax.fori_loop(
        0, _K, topk_step,
        (tuple(jnp.zeros((_G, _K), jnp.int32) for _ in range(_P)), m0))
    for p in range(_P):
        idx_ref[p] = accs[p]


def _run_fps(x4, interpret=False):
    return pl.pallas_call(
        _fps_body,
        out_shape=jax.ShapeDtypeStruct((_B, 8, _L), jnp.float32),
        scratch_shapes=[pltpu.VMEM((_B, _R, _L), jnp.float32)],
        interpret=interpret,
    )(x4)


def _run_topk(xr, ct, interpret=False):
    b = xr.shape[0]
    return pl.pallas_call(
        _topk_body,
        grid=(b // _P,),
        in_specs=[
            pl.BlockSpec((_P, 8, _N), lambda i: (i, 0, 0)),
            pl.BlockSpec((_P, _G, 8), lambda i: (i, 0, 0)),
        ],
        out_specs=pl.BlockSpec((_P, _G, _K), lambda i: (i, 0, 0)),
        out_shape=jax.ShapeDtypeStruct((b, _G, _K), jnp.int32),
        scratch_shapes=[pltpu.VMEM((_P, _G, _N), jnp.float32)],
        interpret=interpret,
    )(xr, ct)


def kernel(xyz):
    b, n, c = xyz.shape
    x_t = jnp.transpose(xyz, (0, 2, 1))
    x4 = x_t.reshape(b, 3, _R, _L)
    xr = jnp.concatenate(
        [x_t, jnp.zeros((b, 8 - c, n), xyz.dtype)], axis=1)
    c_rows = _run_fps(x4)
    ct = jnp.transpose(c_rows, (0, 2, 1))
    center = ct[:, :, :3]
    idx = _run_topk(xr, ct)
    flat = idx.reshape(b, _G * _K)
    patch = jnp.take_along_axis(xyz, flat[:, :, None], axis=1)
    patch = patch.reshape(b, _G, _K, c) - center[:, :, None, :]
    return (patch, center)


# speedup vs baseline: 1.7103x; 1.7103x over previous
import jax
import jax.numpy as jnp
from jax import lax
from jax.experimental import pallas as pl
from jax.experimental.pallas import tpu as pltpu

_B = 16
_G = 128
_K = 32
_R = 64
_L = 128
_N = _R * _L


def _fps_body(x_ref, c_ref, dv_ref):
    iota2 = (lax.broadcasted_iota(jnp.int32, (_R, _L), 0) * _L
             + lax.broadcasted_iota(jnp.int32, (_R, _L), 1))
    subl = lax.broadcasted_iota(jnp.int32, (8, _L), 0)
    lane = lax.broadcasted_iota(jnp.int32, (8, _L), 1)

    dv_ref[:] = jnp.full((_B, _R, _L), 1e10, jnp.float32)

    def _amin2(a):
        return jnp.min(jnp.min(a, axis=0, keepdims=True),
                       axis=1, keepdims=True)

    def _asum2(a):
        return jnp.sum(jnp.sum(a, axis=0, keepdims=True),
                       axis=1, keepdims=True)

    def step(s, fars):
        new_fars = []
        for b in range(_B):
            x0 = x_ref[b, 0]
            x1 = x_ref[b, 1]
            x2 = x_ref[b, 2]
            far = fars[b]
            oh = iota2 == far
            c0 = _asum2(jnp.where(oh, x0, 0.0))
            c1 = _asum2(jnp.where(oh, x1, 0.0))
            c2 = _asum2(jnp.where(oh, x2, 0.0))
            d0 = x0 - c0
            d1 = x1 - c1
            d2 = x2 - c2
            d = d0 * d0 + d1 * d1 + d2 * d2
            dv = jnp.minimum(dv_ref[b], d)
            dv_ref[b] = dv
            m = jnp.max(jnp.max(dv, axis=0, keepdims=True),
                        axis=1, keepdims=True)
            new_fars.append(_amin2(jnp.where(dv == m, iota2, _N)))
            crow = jnp.where(subl == 0, c0, jnp.where(subl == 1, c1, c2))
            c_ref[b] = jnp.where(lane == s, crow, c_ref[b])
        return tuple(new_fars)

    lax.fori_loop(0, _G, step, tuple(jnp.zeros((1, 1), jnp.int32)
                                     for _ in range(_B)))


_P = 4


def _topk_body(xr_ref, c_ref, idx_ref, d_ref):
    for p in range(_P):
        c0a = c_ref[p, :, 0:1]
        c1a = c_ref[p, :, 1:2]
        c2a = c_ref[p, :, 2:3]
        x0r = xr_ref[p, 0:1, :]
        x1r = xr_ref[p, 1:2, :]
        x2r = xr_ref[p, 2:3, :]
        e0 = c0a - x0r
        e1 = c1a - x1r
        e2 = c2a - x2r
        d_ref[p] = jnp.sqrt(e0 * e0 + e1 * e1 + e2 * e2)

    iota_l = lax.broadcasted_iota(jnp.int32, (1, _N), 1)
    iota_k = lax.broadcasted_iota(jnp.int32, (_G, _K), 1)

    m0 = tuple(jnp.min(d_ref[p], axis=1, keepdims=True) for p in range(_P))

    def topk_step(j, carry):
        accs, ms = carry
        new_accs, new_ms = [], []
        for p in range(_P):
            dm = d_ref[p]
            m = ms[p]
            sel = jnp.min(jnp.where(dm == m, iota_l, _N),
                          axis=1, keepdims=True)
            nd = jnp.where(iota_l == sel, jnp.inf, dm)
            d_ref[p] = nd
            new_ms.append(jnp.min(nd, axis=1, keepdims=True))
            new_accs.append(jnp.where(iota_k == j, sel, accs[p]))
        return tuple(new_accs), tuple(new_ms)

    accs, _ = lax.fori_loop(
        0, _K, topk_step,
        (tuple(jnp.zeros((_G, _K), jnp.int32) for _ in range(_P)), m0))
    for p in range(_P):
        idx_ref[p] = accs[p]


def _run_fps(x4, interpret=False):
    return pl.pallas_call(
        _fps_body,
        out_shape=jax.ShapeDtypeStruct((_B, 8, _L), jnp.float32),
        scratch_shapes=[pltpu.VMEM((_B, _R, _L), jnp.float32)],
        interpret=interpret,
    )(x4)


def _run_topk(xr, ct, interpret=False):
    b = xr.shape[0]
    return pl.pallas_call(
        _topk_body,
        grid=(b // _P,),
        in_specs=[
            pl.BlockSpec((_P, 8, _N), lambda i: (i, 0, 0)),
            pl.BlockSpec((_P, _G, 8), lambda i: (i, 0, 0)),
        ],
        out_specs=pl.BlockSpec((_P, _G, _K), lambda i: (i, 0, 0)),
        out_shape=jax.ShapeDtypeStruct((b, _G, _K), jnp.int32),
        scratch_shapes=[pltpu.VMEM((_P, _G, _N), jnp.float32)],
        interpret=interpret,
    )(xr, ct)


def kernel(xyz):
    b, n, c = xyz.shape
    x_t = jnp.transpose(xyz, (0, 2, 1))
    x4 = x_t.reshape(b, 3, _R, _L)
    xr = jnp.concatenate(
        [x_t, jnp.zeros((b, 8 - c, n), xyz.dtype)], axis=1)
    c_rows = _run_fps(x4)
    ct = jnp.transpose(c_rows, (0, 2, 1))
    center = ct[:, :, :3]
    idx = _run_topk(xr, ct)
    flat = idx.reshape(b, _G * _K)
    patch = jnp.take_along_axis(xyz, flat[:, :, None], axis=1)
    patch = patch.reshape(b, _G, _K, c) - center[:, :, None, :]
    return (patch, center)
